# Initial kernel scaffold; baseline (speedup 1.0000x reference)
import functools

import numpy as np
import jax
import jax.numpy as jnp
from jax import lax
from jax.experimental import pallas as pl
from jax.experimental.pallas import tpu as pltpu
from jax.experimental.pallas import tpu_sc as plsc

_B, _T = 16, 32768
_NW = 32
_C = _T // _NW
_P = 1104
_W = _P + _C
_L = 16

_SAMPLE_RATE = 44100
_NUM_VOICES = 4
_RATE = 1.5


def _local_columns() -> np.ndarray:
    base_delay = int(20.0 * _SAMPLE_RATE / 1000)
    range_samples = int(10.0 * _SAMPLE_RATE / 1000 * 0.5)
    tf = np.arange(_T, dtype=np.float64)
    ti = np.arange(_T, dtype=np.int64)
    cols = np.empty((_NUM_VOICES, _T), dtype=np.int32)
    for i in range(_NUM_VOICES):
        phase = (i / _NUM_VOICES + tf * _RATE / _SAMPLE_RATE) % 1.0
        mod = np.sin(2 * np.pi * phase)
        delay = base_delay + np.trunc(mod * range_samples).astype(np.int64)
        delay = np.clip(delay, 1, 2047)
        cols[i] = (_P + (ti % _C) - delay).astype(np.int32)
    return cols


_LIDX = _local_columns()


def _chorus_sc(xp, lidx):
    mesh = plsc.VectorSubcoreMesh(core_axis_name="c", subcore_axis_name="s")

    @functools.partial(
        pl.kernel,
        mesh=mesh,
        out_type=jax.ShapeDtypeStruct((_B, _T), jnp.float32),
        scratch_types=[
            pltpu.VMEM((_B, _W), jnp.float32),
            pltpu.VMEM((_NUM_VOICES, _C), jnp.int32),
            pltpu.VMEM((_B, _C), jnp.float32),
            pltpu.SemaphoreType.DMA,
        ],
    )
    def k(xp_hbm, lidx_hbm, out_hbm, xw, idxv, outv, sem):
        nc = 2
        wid = lax.axis_index("s") * nc + lax.axis_index("c")
        t0 = wid * _C

        handles = []
        for b in range(_B):
            handles.append(
                pltpu.async_copy(xp_hbm.at[b, pl.ds(t0, _W)], xw.at[b], sem))
        for i in range(_NUM_VOICES):
            handles.append(
                pltpu.async_copy(lidx_hbm.at[i, pl.ds(t0, _C)], idxv.at[i], sem))
        for h in handles:
            h.wait()

        def body(v, carry):
            base = v * _L
            ivs = [idxv[i, pl.ds(base, _L)] for i in range(_NUM_VOICES)]
            for b in range(_B):
                row = jnp.full((_L,), b, jnp.int32)
                g = plsc.load_gather(xw, [row, ivs[0]])
                for i in range(1, _NUM_VOICES):
                    g = g + plsc.load_gather(xw, [row, ivs[i]])
                dry = xw[b, pl.ds(_P + base, _L)]
                outv[b, pl.ds(base, _L)] = dry * 0.5 + g * 0.125
            return carry

        lax.fori_loop(0, _C // _L, body, 0)

        out_handles = []
        for b in range(_B):
            out_handles.append(
                pltpu.async_copy(outv.at[b], out_hbm.at[b, pl.ds(t0, _C)], sem))
        for h in out_handles:
            h.wait()

    return k(xp, jnp.asarray(lidx))


def kernel(x):
    xp = jnp.pad(x, ((0, 0), (_P, 0)))
    return _chorus_sc(xp, _LIDX)


# speedup vs baseline: 9.8138x; 9.8138x over previous
import functools

import numpy as np
import jax
import jax.numpy as jnp
from jax import lax
from jax.experimental import pallas as pl
from jax.experimental.pallas import tpu as pltpu
from jax.experimental.pallas import tpu_sc as plsc

_B, _T = 16, 32768
_NW = 32
_C = _T // _NW
_P = 1104
_Z = 16
_W = _Z + _P + _C
_L = 16

_SAMPLE_RATE = 44100
_NUM_VOICES = 4
_RATE = 1.5


def _local_columns() -> np.ndarray:
    base_delay = int(20.0 * _SAMPLE_RATE / 1000)
    range_samples = int(10.0 * _SAMPLE_RATE / 1000 * 0.5)
    tf = np.arange(_T, dtype=np.float64)
    ti = np.arange(_T, dtype=np.int64)
    cols = np.empty((_NUM_VOICES, _T), dtype=np.int32)
    for i in range(_NUM_VOICES):
        phase = (i / _NUM_VOICES + tf * _RATE / _SAMPLE_RATE) % 1.0
        mod = np.sin(2 * np.pi * phase)
        delay = base_delay + np.trunc(mod * range_samples).astype(np.int64)
        delay = np.clip(delay, 1, 2047)
        col = _Z + _P + (ti % _C) - delay
        cols[i] = np.where(ti >= delay, col, 0).astype(np.int32)
    return cols


_LIDX = _local_columns()


def _chorus_sc(xf, lidx):
    mesh = plsc.VectorSubcoreMesh(core_axis_name="c", subcore_axis_name="s")

    @functools.partial(
        pl.kernel,
        mesh=mesh,
        compiler_params=pltpu.CompilerParams(needs_layout_passes=False),
        out_type=jax.ShapeDtypeStruct((_B * _T,), jnp.float32),
        scratch_types=[
            pltpu.VMEM((_B * _W,), jnp.float32),
            pltpu.VMEM((_NUM_VOICES * _C,), jnp.int32),
            pltpu.VMEM((_B * _C,), jnp.float32),
            pltpu.SemaphoreType.DMA,
        ],
    )
    def k(x_hbm, lidx_hbm, out_hbm, xw, idxv, outv, sem):
        nc = 2
        wid = lax.axis_index("s") * nc + lax.axis_index("c")
        t0 = wid * _C

        zero = jnp.zeros((_L,), jnp.float32)
        for b in range(_B):
            xw[pl.ds(b * _W, _L)] = zero

        idx_handles = [
            pltpu.async_copy(lidx_hbm.at[pl.ds(i * _T + t0, _C)],
                             idxv.at[pl.ds(i * _C, _C)], sem)
            for i in range(_NUM_VOICES)
        ]

        @pl.when(wid == 0)
        def _():
            hs = [pltpu.async_copy(x_hbm.at[pl.ds(b * _T, _C)],
                                   xw.at[pl.ds(b * _W + _Z + _P, _C)], sem)
                  for b in range(_B)]
            for h in hs:
                h.wait()

        @pl.when(wid == 1)
        def _():
            hs = [pltpu.async_copy(x_hbm.at[pl.ds(b * _T, 2 * _C)],
                                   xw.at[pl.ds(b * _W + _Z + (_P - _C), 2 * _C)],
                                   sem)
                  for b in range(_B)]
            for h in hs:
                h.wait()

        @pl.when(wid >= 2)
        def _():
            hs = [pltpu.async_copy(x_hbm.at[pl.ds(b * _T + t0 - _P, _P + _C)],
                                   xw.at[pl.ds(b * _W + _Z, _P + _C)], sem)
                  for b in range(_B)]
            for h in hs:
                h.wait()

        for h in idx_handles:
            h.wait()

        def body(v, carry):
            base = v * _L
            ivs = [idxv[pl.ds(i * _C + base, _L)] for i in range(_NUM_VOICES)]
            for b in range(_B):
                off = b * _W
                g = plsc.load_gather(xw, [ivs[0] + off])
                for i in range(1, _NUM_VOICES):
                    g = g + plsc.load_gather(xw, [ivs[i] + off])
                dry = xw[pl.ds(off + _Z + _P + base, _L)]
                outv[pl.ds(b * _C + base, _L)] = dry * 0.5 + g * 0.125
            return carry

        lax.fori_loop(0, _C // _L, body, 0)

        out_handles = [
            pltpu.async_copy(outv.at[pl.ds(b * _C, _C)],
                             out_hbm.at[pl.ds(b * _T + t0, _C)], sem)
            for b in range(_B)
        ]
        for h in out_handles:
            h.wait()

    return k(xf, jnp.asarray(lidx).reshape(-1))


def kernel(x):
    return _chorus_sc(x.reshape(-1), _LIDX).reshape(_B, _T)


# speedup vs baseline: 11.8877x; 1.2113x over previous
import functools

import numpy as np
import jax
import jax.numpy as jnp
from jax import lax
from jax.experimental import pallas as pl
from jax.experimental.pallas import tpu as pltpu
from jax.experimental.pallas import tpu_sc as plsc

_B, _T = 16, 32768
_NW = 32
_C = _T // _NW
_P = 1104
_Z = 16
_W = _Z + _P + _C
_L = 16

_SAMPLE_RATE = 44100
_NUM_VOICES = 4
_RATE = 1.5


def _local_columns() -> np.ndarray:
    base_delay = int(20.0 * _SAMPLE_RATE / 1000)
    range_samples = int(10.0 * _SAMPLE_RATE / 1000 * 0.5)
    tf = np.arange(_T, dtype=np.float64)
    ti = np.arange(_T, dtype=np.int64)
    cols = np.empty((_NUM_VOICES, _T), dtype=np.int32)
    for i in range(_NUM_VOICES):
        phase = (i / _NUM_VOICES + tf * _RATE / _SAMPLE_RATE) % 1.0
        mod = np.sin(2 * np.pi * phase)
        delay = base_delay + np.trunc(mod * range_samples).astype(np.int64)
        delay = np.clip(delay, 1, 2047)
        col = _Z + _P + (ti % _C) - delay
        cols[i] = np.where(ti >= delay, col, 0).astype(np.int32)
    return cols


_LIDX = _local_columns()


def _chorus_sc(x, lidx):
    mesh = plsc.VectorSubcoreMesh(core_axis_name="c", subcore_axis_name="s")

    @functools.partial(
        pl.kernel,
        mesh=mesh,
        compiler_params=pltpu.CompilerParams(
            needs_layout_passes=False, use_tc_tiling_on_sc=False),
        out_type=jax.ShapeDtypeStruct((_B, _T), jnp.float32),
        scratch_types=[
            pltpu.VMEM((_B, _W), jnp.float32),
            pltpu.VMEM((_NUM_VOICES, _C), jnp.int32),
            pltpu.VMEM((_B, _C), jnp.float32),
            pltpu.SemaphoreType.DMA,
        ],
    )
    def k(x_hbm, lidx_hbm, out_hbm, xw, idxv, outv, sem):
        nc = 2
        wid = lax.axis_index("s") * nc + lax.axis_index("c")
        t0 = wid * _C

        zero = jnp.zeros((_L,), jnp.float32)
        for b in range(_B):
            xw[b, pl.ds(0, _L)] = zero

        h_idx = pltpu.async_copy(lidx_hbm.at[:, pl.ds(t0, _C)], idxv, sem)

        @pl.when(wid == 0)
        def _():
            pltpu.async_copy(
                x_hbm.at[:, pl.ds(0, _C)],
                xw.at[:, pl.ds(_Z + _P, _C)], sem).wait()

        @pl.when(wid == 1)
        def _():
            pltpu.async_copy(
                x_hbm.at[:, pl.ds(0, 2 * _C)],
                xw.at[:, pl.ds(_Z + (_P - _C), 2 * _C)], sem).wait()

        @pl.when(wid >= 2)
        def _():
            pltpu.async_copy(
                x_hbm.at[:, pl.ds(t0 - _P, _P + _C)],
                xw.at[:, pl.ds(_Z, _P + _C)], sem).wait()

        h_idx.wait()

        rows = [jnp.full((_L,), b, jnp.int32) for b in range(_B)]

        @plsc.parallel_loop(0, _C // _L)
        def _body(v):
            base = v * _L
            ivs = [idxv[i, pl.ds(base, _L)] for i in range(_NUM_VOICES)]
            for b in range(_B):
                g = plsc.load_gather(xw, [rows[b], ivs[0]])
                for i in range(1, _NUM_VOICES):
                    g = g + plsc.load_gather(xw, [rows[b], ivs[i]])
                dry = xw[b, pl.ds(_Z + _P + base, _L)]
                outv[b, pl.ds(base, _L)] = dry * 0.5 + g * 0.125

        pltpu.async_copy(outv, out_hbm.at[:, pl.ds(t0, _C)], sem).wait()

    return k(x, jnp.asarray(lidx))


def kernel(x):
    return _chorus_sc(x, _LIDX)


# speedup vs baseline: 14.1691x; 1.1919x over previous
import functools

import numpy as np
import jax
import jax.numpy as jnp
from jax import lax
from jax.experimental import pallas as pl
from jax.experimental.pallas import tpu as pltpu
from jax.experimental.pallas import tpu_sc as plsc

_B, _T = 16, 32768
_NW = 32
_C = _T // _NW
_P = 1152
_W = _P + _C
_L = 16

_SAMPLE_RATE = 44100
_NUM_VOICES = 4
_RATE = 1.5


def _local_columns() -> np.ndarray:
    base_delay = int(20.0 * _SAMPLE_RATE / 1000)
    range_samples = int(10.0 * _SAMPLE_RATE / 1000 * 0.5)
    tf = np.arange(_T, dtype=np.float64)
    ti = np.arange(_T, dtype=np.int64)
    cols = np.empty((_NUM_VOICES, _T), dtype=np.int32)
    for i in range(_NUM_VOICES):
        phase = (i / _NUM_VOICES + tf * _RATE / _SAMPLE_RATE) % 1.0
        mod = np.sin(2 * np.pi * phase)
        delay = base_delay + np.trunc(mod * range_samples).astype(np.int64)
        delay = np.clip(delay, 1, 2047)
        col = _P + (ti % _C) - delay
        cols[i] = np.where(ti >= delay, col, ti % _L).astype(np.int32)
    return cols


_LIDX = _local_columns()


def _chorus_sc(x, lidx):
    mesh = plsc.VectorSubcoreMesh(core_axis_name="c", subcore_axis_name="s")

    @functools.partial(
        pl.kernel,
        mesh=mesh,
        compiler_params=pltpu.CompilerParams(needs_layout_passes=False),
        out_type=jax.ShapeDtypeStruct((_B, _T), jnp.float32),
        scratch_types=[
            pltpu.VMEM((_B, _W), jnp.float32),
            pltpu.VMEM((_NUM_VOICES * _C,), jnp.int32),
            pltpu.VMEM((_B, _C), jnp.float32),
            pltpu.SemaphoreType.DMA,
        ],
    )
    def k(x_hbm, lidx_hbm, out_hbm, xw, idxv, outv, sem):
        nc = 2
        wid = lax.axis_index("s") * nc + lax.axis_index("c")
        t0 = wid * _C

        h_idx = [
            pltpu.async_copy(lidx_hbm.at[pl.ds(i * _T + t0, _C)],
                             idxv.at[pl.ds(i * _C, _C)], sem)
            for i in range(_NUM_VOICES)
        ]

        zero = jnp.zeros((_L,), jnp.float32)

        @pl.when(wid == 0)
        def _():
            for b in range(_B):
                xw[b, pl.ds(0, _L)] = zero
            pltpu.async_copy(
                x_hbm.at[:, pl.ds(0, _C)],
                xw.at[:, pl.ds(_P, _C)], sem).wait()

        @pl.when(wid == 1)
        def _():
            for b in range(_B):
                xw[b, pl.ds(0, _L)] = zero
            pltpu.async_copy(
                x_hbm.at[:, pl.ds(0, 2 * _C)],
                xw.at[:, pl.ds(_P - _C, 2 * _C)], sem).wait()

        @pl.when(wid >= 2)
        def _():
            pltpu.async_copy(
                x_hbm.at[:, pl.ds(t0 - _P, _W)],
                xw.at[:, pl.ds(0, _W)], sem).wait()

        for h in h_idx:
            h.wait()

        rows = [jnp.full((_L,), b, jnp.int32) for b in range(_B)]

        @plsc.parallel_loop(0, _C // _L)
        def _body(v):
            base = v * _L
            ivs = [idxv[pl.ds(i * _C + base, _L)] for i in range(_NUM_VOICES)]
            for b in range(_B):
                g = plsc.load_gather(xw, [rows[b], ivs[0]])
                for i in range(1, _NUM_VOICES):
                    g = g + plsc.load_gather(xw, [rows[b], ivs[i]])
                dry = xw[b, pl.ds(_P + base, _L)]
                outv[b, pl.ds(base, _L)] = dry * 0.5 + g * 0.125

        pltpu.async_copy(outv, out_hbm.at[:, pl.ds(t0, _C)], sem).wait()

    return k(x, jnp.asarray(lidx).reshape(-1))


def kernel(x):
    return _chorus_sc(x, _LIDX)
